# baseline (speedup 1.0000x reference)
import functools
import math

import jax
import jax.numpy as jnp
from jax import lax
from jax.experimental import pallas as pl
from jax.experimental.pallas import tpu as pltpu
from jax.experimental.pallas import tpu_sc as plsc

DEPTH = 64
SCALE = math.sqrt(DEPTH)
VOCAB_N = 1000000

NC = 2
NS = 16
NW = NC * NS
LANES = 16
BB = 128
NT = 200
NBUF = 2

N_FULL = VOCAB_N // BB
TAIL = VOCAB_N - N_FULL * BB
N_UNITS = N_FULL + 1
A_ITERS = (N_UNITS + NW - 1) // NW


def _make_relayout():
  mesh = plsc.VectorSubcoreMesh(core_axis_name="c", subcore_axis_name="s")

  @functools.partial(
      pl.kernel,
      mesh=mesh,
      out_type=jax.ShapeDtypeStruct((VOCAB_N // 2, BB), jnp.float32),
      scratch_types=[
          [pltpu.VMEM((DEPTH, BB), jnp.float32) for _ in range(NBUF)],
          [pltpu.VMEM((DEPTH, BB), jnp.float32) for _ in range(NBUF)],
          [pltpu.SemaphoreType.DMA for _ in range(NBUF)],
          [pltpu.SemaphoreType.DMA for _ in range(NBUF)],
      ],
      compiler_params=pltpu.CompilerParams(
          use_tc_tiling_on_sc=True, needs_layout_passes=False),
  )
  def relayout(lutt_hbm, tail_hbm, r_hbm, sbufs, obufs, gsems, ssems):
    w = lax.axis_index("s") * NC + lax.axis_index("c")
    iota = lax.iota(jnp.int32, 16)
    n_steps = (A_ITERS + NBUF - 1) // NBUF

    def unit(i):
      return lax.rem(w + i * NW, N_FULL)

    def load_slab(u, b):
      return pltpu.make_async_copy(
          lutt_hbm.at[:, pl.ds(u * BB, BB)], sbufs[b], gsems[b])

    def store(u, b):
      return pltpu.make_async_copy(
          obufs[b].at[pl.ds(0, BB // 2)],
          r_hbm.at[pl.ds(u * (BB // 2), BB // 2)], ssems[b])

    def transpose(b, nrows):
      @plsc.parallel_loop(0, nrows, unroll=4)
      def _(r):
        for m in range(8):
          col = 2 * r + (m // 4)
          v = plsc.load_gather(
              sbufs[b], [iota + (m % 4) * LANES, iota * 0 + col])
          obufs[b][r, pl.ds((m % 4) * LANES + (m // 4) * DEPTH, LANES)] = (
              v * SCALE)

    for b in range(NBUF):
      load_slab(unit(b), b).start()

    def step(i, carry):
      for b in range(NBUF):
        u = unit(i * NBUF + b)
        load_slab(0, b).wait()

        @pl.when(i > 0)
        def _():
          store(0, b).wait()

        transpose(b, BB // 2)
        load_slab(unit((i + 1) * NBUF + b), b).start()
        store(u, b).start()
      return carry

    lax.fori_loop(0, n_steps, step, 0)
    for b in range(NBUF):
      load_slab(0, b).wait()
      store(0, b).wait()

    @pl.when(w == 0)
    def _():
      pltpu.sync_copy(tail_hbm, sbufs[0])
      transpose(0, TAIL // 2)
      pltpu.sync_copy(
          obufs[0].at[pl.ds(0, TAIL // 2)],
          r_hbm.at[pl.ds(N_FULL * (BB // 2), TAIL // 2)])

  return relayout


def _make_lookup():
  mesh = plsc.VectorSubcoreMesh(core_axis_name="c", subcore_axis_name="s")

  @functools.partial(
      pl.kernel,
      mesh=mesh,
      out_type=jax.ShapeDtypeStruct((NT, 8, NW, 8, BB), jnp.float32),
      scratch_types=[
          pltpu.VMEM((BB, NT), jnp.int32),
          [pltpu.VMEM((BB,), jnp.int32) for _ in range(NBUF)],
          [pltpu.VMEM((BB, DEPTH), jnp.float32) for _ in range(NBUF)],
          [pltpu.VMEM((DEPTH, BB), jnp.float32) for _ in range(NBUF)],
          [pltpu.SemaphoreType.DMA for _ in range(NBUF)],
          [pltpu.SemaphoreType.DMA for _ in range(NBUF)],
      ],
      compiler_params=pltpu.CompilerParams(
          use_tc_tiling_on_sc=False, needs_layout_passes=False),
  )
  def lookup(lut_hbm, x_hbm, out_hbm, xs, ibufs, gbufs, obufs, gsems, ssems):
    j = lax.axis_index("s") * NC + lax.axis_index("c")
    pltpu.sync_copy(x_hbm.at[pl.ds(j * BB, BB)], xs)

    iota = lax.iota(jnp.int32, 16)

    def extract_idx(t, b):
      for bb in range(BB // LANES):
        ibufs[b][pl.ds(bb * LANES, LANES)] = plsc.load_gather(
            xs, [iota + bb * LANES, iota * 0 + t])

    def gather(b):
      return pltpu.make_async_copy(lut_hbm.at[ibufs[b]], gbufs[b], gsems[b])

    def store_start(t, b):
      for dblk in range(DEPTH // 8):
        pltpu.make_async_copy(
            obufs[b].at[pl.ds(dblk * 8, 8)], out_hbm.at[t, dblk, j],
            ssems[b]).start()

    def store_wait(b):
      for dblk in range(DEPTH // 8):
        pltpu.make_async_copy(
            obufs[b].at[pl.ds(dblk * 8, 8)], out_hbm.at[0, dblk, j],
            ssems[b]).wait()

    for b in range(NBUF):
      extract_idx(b, b)
      gather(b).start()

    def do_pair(g, carry):
      for b in range(NBUF):
        t = g * NBUF + b
        gather(b).wait()

        @pl.when(g > 0)
        def _():
          store_wait(b)

        @plsc.parallel_loop(0, DEPTH, unroll=4)
        def _(d):
          for bb in range(BB // LANES):
            obufs[b][d, pl.ds(bb * LANES, LANES)] = plsc.load_gather(
                gbufs[b], [iota + bb * LANES, iota * 0 + d])

        @pl.when(t + NBUF < NT)
        def _():
          extract_idx(t + NBUF, b)
          gather(b).start()

        store_start(t, b)
      return carry

    lax.fori_loop(0, NT // NBUF, do_pair, 0)

    for b in range(NBUF):
      store_wait(b)

  return lookup


def kernel(x, lut):
  tail = jnp.pad(jnp.transpose(lut[N_FULL * BB:, :]), ((0, 0), (0, BB - TAIL)))
  r = _make_relayout()(jnp.transpose(lut), tail)
  lut_rm = r.reshape(VOCAB_N, DEPTH)
  out5 = _make_lookup()(lut_rm, x.astype(jnp.int32))
  return out5.transpose(2, 4, 0, 1, 3).reshape(4096, 200, 64)
